# Initial kernel scaffold; baseline (speedup 1.0000x reference)
import jax
import jax.numpy as jnp
from jax.experimental import pallas as pl


def kernel(x, edge_index, batch, node_coords, params):
    raise NotImplementedError("write your pallas kernel here")


# speedup vs baseline: 3.5894x; 3.5894x over previous
import functools

import jax
import jax.numpy as jnp
from jax import lax
from jax.experimental import pallas as pl
from jax.experimental.pallas import tpu as pltpu
from jax.experimental.pallas import tpu_sc as plsc

N_NODES = 10000
N_EDGES = 160000
INPUT_DIM = 256
EMB_DIM = 64
NUM_LAYER = 5
NUM_CLASS = 10
NUM_GRAPHS = 128

NC = 2
NS = 16
NW = NC * NS

CHUNK = 128
CPT = 40
E_PAD = NW * CPT * CHUNK
N_PAD = 10112
ROWS_PER_TILE = N_PAD // NS


def _make_segsum(D):
    mesh = plsc.VectorSubcoreMesh(core_axis_name="c", subcore_axis_name="s")

    @functools.partial(
        pl.kernel,
        out_type=jax.ShapeDtypeStruct((NC, N_PAD, D), jnp.float32),
        mesh=mesh,
        scratch_types=[
            pltpu.VMEM((CPT, CHUNK), jnp.int32),
            pltpu.VMEM((CPT, CHUNK), jnp.int32),
            pltpu.VMEM((CHUNK, D), jnp.float32),
            pltpu.VMEM_SHARED((N_PAD, D), jnp.float32),
            pltpu.SemaphoreType.DMA,
        ],
        compiler_params=pltpu.CompilerParams(use_tc_tiling_on_sc=False),
    )
    def segsum(h_hbm, srcs_hbm, dsts_hbm, zeros_hbm, out_hbm,
               src_v, dst_v, rows_v, agg_sh, sem):
        c = lax.axis_index("c")
        s = lax.axis_index("s")
        wid = s * NC + c
        base = s * ROWS_PER_TILE
        pltpu.sync_copy(zeros_hbm.at[pl.ds(base, ROWS_PER_TILE)],
                        agg_sh.at[pl.ds(base, ROWS_PER_TILE)])
        pltpu.sync_copy(srcs_hbm.at[wid], src_v)
        pltpu.sync_copy(dsts_hbm.at[wid], dst_v)
        plsc.subcore_barrier()

        @pl.loop(0, CPT)
        def _(j):
            pltpu.async_copy(h_hbm.at[src_v.at[j]], rows_v, sem).wait()
            pltpu.sync_copy(rows_v, agg_sh.at[dst_v.at[j]], add=True)

        plsc.subcore_barrier()
        pltpu.sync_copy(agg_sh.at[pl.ds(base, ROWS_PER_TILE)],
                        out_hbm.at[c, pl.ds(base, ROWS_PER_TILE)])

    return segsum


_segsum_128 = _make_segsum(2 * EMB_DIM)
_segsum_64 = _make_segsum(EMB_DIM)


def _rowmask():
    return lax.broadcasted_iota(jnp.int32, (N_PAD, 1), 0) < N_NODES


def _bn_relu(m, gamma, beta, relu):
    mask = _rowmask()
    m = jnp.where(mask, m, 0.0)
    mean = jnp.sum(m, axis=0, keepdims=True) * (1.0 / N_NODES)
    cent = jnp.where(mask, m - mean, 0.0)
    var = jnp.sum(cent * cent, axis=0, keepdims=True) * (1.0 / N_NODES)
    z = cent / jnp.sqrt(var + 1e-5) * gamma + beta
    z = jnp.where(mask, z, 0.0)
    if relu:
        z = jnp.maximum(z, 0.0)
    return z


def _mlp_bn(z, w1_ref, b1_ref, w2_ref, b2_ref, g_ref, bt_ref, relu):
    t = jnp.maximum(jnp.dot(z, w1_ref[...], preferred_element_type=jnp.float32)
                    + b1_ref[...], 0.0)
    m = jnp.dot(t, w2_ref[...], preferred_element_type=jnp.float32) + b2_ref[...]
    return _bn_relu(m, g_ref[...], bt_ref[...], relu=relu)


def _layer0_body(x_ref, aggl_ref, aggr_ref, w1_ref, b1_ref, w2_ref, b2_ref,
                 g_ref, bt_ref, eps_ref, o_ref):
    agg = jnp.concatenate([aggl_ref[0] + aggl_ref[1], aggr_ref[0] + aggr_ref[1]],
                          axis=1)
    z = (1.0 + eps_ref[0, 0]) * x_ref[...] + agg
    o_ref[...] = _mlp_bn(z, w1_ref, b1_ref, w2_ref, b2_ref, g_ref, bt_ref, True)


def _layer_body(h_ref, agg_ref, w1_ref, b1_ref, w2_ref, b2_ref, g_ref, bt_ref,
                eps_ref, o_ref):
    z = (1.0 + eps_ref[0, 0]) * h_ref[...] + agg_ref[0] + agg_ref[1]
    o_ref[...] = _mlp_bn(z, w1_ref, b1_ref, w2_ref, b2_ref, g_ref, bt_ref, True)


def _last_body(h_ref, agg_ref, w1_ref, b1_ref, w2_ref, b2_ref, g_ref, bt_ref,
               eps_ref, batch_ref, wh1_ref, wh2_ref, o_ref):
    z = (1.0 + eps_ref[0, 0]) * h_ref[...] + agg_ref[0] + agg_ref[1]
    h = _mlp_bn(z, w1_ref, b1_ref, w2_ref, b2_ref, g_ref, bt_ref, False)
    gid = lax.broadcasted_iota(jnp.int32, (N_PAD, NUM_GRAPHS), 1)
    onehot = (batch_ref[...] == gid).astype(jnp.float32)
    sums = lax.dot_general(onehot, h, (((0,), (0,)), ((), ())),
                           preferred_element_type=jnp.float32,
                           precision=lax.Precision.HIGHEST)
    counts = jnp.sum(onehot, axis=0, keepdims=True)
    hg = sums / jnp.maximum(counts, 1.0).T
    o_ref[...] = jnp.dot(jnp.dot(hg, wh1_ref[...], preferred_element_type=jnp.float32),
                         wh2_ref[...], preferred_element_type=jnp.float32)


def kernel(x, edge_index, batch, node_coords, params):
    del node_coords
    f32 = jnp.float32
    src = edge_index[0]
    dst = edge_index[1]
    src_p = jnp.concatenate(
        [src, jnp.zeros((E_PAD - N_EDGES,), jnp.int32)]).reshape(NW, CPT, CHUNK)
    dst_p = jnp.concatenate(
        [dst, jnp.full((E_PAD - N_EDGES,), N_NODES, jnp.int32)]).reshape(NW, CPT, CHUNK)
    x_p = jnp.pad(x, ((0, N_PAD - N_NODES), (0, 0)))
    batch_p = jnp.pad(batch, (0, N_PAD - N_NODES),
                      constant_values=NUM_GRAPHS).reshape(N_PAD, 1)
    z128 = jnp.zeros((N_PAD, 2 * EMB_DIM), f32)
    z64 = jnp.zeros((N_PAD, EMB_DIM), f32)

    P = params
    r1 = lambda a: a.reshape(1, -1)
    eps = [P["eps"][i].reshape(1, 1) for i in range(NUM_LAYER)]

    aggl = _segsum_128(x_p[:, :128], src_p, dst_p, z128)
    aggr = _segsum_128(x_p[:, 128:], src_p, dst_p, z128)
    h = pl.pallas_call(
        _layer0_body,
        out_shape=jax.ShapeDtypeStruct((N_PAD, EMB_DIM), f32),
    )(x_p, aggl, aggr, P["W1"][0], r1(P["b1"][0]), P["W2"][0], r1(P["b2"][0]),
      r1(P["gamma"][0]), r1(P["beta"][0]), eps[0])

    for i in range(1, NUM_LAYER - 1):
        agg = _segsum_64(h, src_p, dst_p, z64)
        h = pl.pallas_call(
            _layer_body,
            out_shape=jax.ShapeDtypeStruct((N_PAD, EMB_DIM), f32),
        )(h, agg, P["W1"][i], r1(P["b1"][i]), P["W2"][i], r1(P["b2"][i]),
          r1(P["gamma"][i]), r1(P["beta"][i]), eps[i])

    i = NUM_LAYER - 1
    agg = _segsum_64(h, src_p, dst_p, z64)
    out = pl.pallas_call(
        _last_body,
        out_shape=jax.ShapeDtypeStruct((NUM_GRAPHS, NUM_CLASS), f32),
    )(h, agg, P["W1"][i], r1(P["b1"][i]), P["W2"][i], r1(P["b2"][i]),
      r1(P["gamma"][i]), r1(P["beta"][i]), eps[i], batch_p, P["Wh1"], P["Wh2"])
    return out
